# baseline (speedup 1.0000x reference)
import functools

import jax
import jax.numpy as jnp
from jax import lax
from jax.experimental import pallas as pl
from jax.experimental.pallas import tpu as pltpu
from jax.experimental.pallas import tpu_sc as plsc

V = 30000
D = 256
H = 256
B = 16
U = 15
W = 30
NROW = B * (U + 1)
P = NROW * W
NW = 32
RPW = P // NW
NPW = NROW // NW
CHUNK = 80
WPC = CHUNK // NPW


def _sc_gather(contexts, queries, cul, ql, ctx_len, emb):
    mesh = plsc.VectorSubcoreMesh(core_axis_name="c", subcore_axis_name="s")

    @functools.partial(
        pl.kernel,
        out_type=(jax.ShapeDtypeStruct((W, NROW, D), jnp.float32),
                  jax.ShapeDtypeStruct((NROW,), jnp.int32)),
        mesh=mesh,
        compiler_params=pltpu.CompilerParams(needs_layout_passes=False),
        scratch_types=[
            pltpu.VMEM((7296,), jnp.int32),
            pltpu.VMEM((512,), jnp.int32),
            pltpu.VMEM((256,), jnp.int32),
            pltpu.VMEM((128,), jnp.int32),
            pltpu.VMEM((128,), jnp.int32),
            pltpu.VMEM((RPW,), jnp.int32),
            pltpu.VMEM((RPW, D), jnp.float32),
            pltpu.VMEM((NROW,), jnp.int32),
            pltpu.SemaphoreType.DMA,
            pltpu.SemaphoreType.DMA,
        ],
    )
    def sc_kernel(ctx_hbm, q_hbm, cul_hbm, ql_hbm, len_hbm, emb_hbm,
                  x_hbm, lens_hbm,
                  ctx_v, q_v, cul_v, ql_v, len_v, idx_v, rows_v, lens_v,
                  gsem, osem):
        wid = lax.axis_index("s") * 2 + lax.axis_index("c")
        n0 = wid * NPW
        pltpu.sync_copy(ctx_hbm, ctx_v.at[pl.ds(0, B * U * W)])
        pltpu.sync_copy(q_hbm, q_v.at[pl.ds(0, B * W)])
        pltpu.sync_copy(len_hbm, len_v.at[pl.ds(0, B)])
        lane = lax.iota(jnp.int32, 16)
        us = n0 >> 4
        b = (n0 & 15) + (lane & 7)
        lb = plsc.load_gather(len_v, [b])
        su = jnp.maximum(jnp.where(us < lb, us, us - 1), 0)
        is_q = lb == us
        coff = b * (U * W) + su * W
        qoff = b * W
        wbase = lane >> 3
        out_descs = []
        for c in range(RPW // CHUNK):
            for i in range(CHUNK // 16):
                w = c * WPC + 2 * i + wbase
                tok_c = plsc.load_gather(ctx_v, [coff + w])
                tok_q = plsc.load_gather(q_v, [qoff + w])
                idx_v[pl.ds(c * CHUNK + i * 16, 16)] = jnp.where(
                    is_q, tok_q, tok_c)
            gd = pltpu.async_copy(
                emb_hbm.at[idx_v.at[pl.ds(c * CHUNK, CHUNK)]],
                rows_v.at[pl.ds(c * CHUNK, CHUNK)],
                gsem,
            )
            if c > 0:
                for w in range((c - 1) * WPC, c * WPC):
                    out_descs.append(pltpu.async_copy(
                        rows_v.at[pl.ds(w * NPW, NPW)],
                        x_hbm.at[w, pl.ds(n0, NPW)],
                        osem,
                    ))
            gd.wait()
        for w in range((RPW // CHUNK - 1) * WPC, W):
            out_descs.append(pltpu.async_copy(
                rows_v.at[pl.ds(w * NPW, NPW)],
                x_hbm.at[w, pl.ds(n0, NPW)],
                osem,
            ))
        @pl.when(wid == 0)
        def _():
            pltpu.sync_copy(cul_hbm, cul_v.at[pl.ds(0, B * U)])
            pltpu.sync_copy(ql_hbm, ql_v.at[pl.ds(0, B)])
            for u in range(U + 1):
                if u < U:
                    lv = plsc.load_gather(cul_v, [lane * U + u])
                else:
                    lv = plsc.load_gather(ql_v, [lane])
                lens_v[pl.ds(u * B, 16)] = lv
            pltpu.sync_copy(lens_v, lens_hbm)
        for d in out_descs:
            d.wait()

    return sc_kernel(contexts, queries, cul, ql, ctx_len, emb)


def _tc_gru(x_tm, lens, ctx_len, wx_u, wh_u, b_u, wx_c, wh_c, b_c):
    def sg(a):
        return 0.5 + 0.5 * jnp.tanh(0.5 * a)

    def tc_kernel(x_ref, len_ref, cl_ref, wxu_ref, whu_ref, bu_ref, wxc_ref,
                  whc_ref, bc_ref, out_ref, gx_ref, g2_ref):
        whu = whu_ref[...]
        lenv = len_ref[...]

        x2d = x_ref[...].reshape(W * NROW, D)
        gx_ref[...] = (jnp.dot(x2d, wxu_ref[...],
                               preferred_element_type=jnp.float32) + bu_ref[...])

        h = jnp.zeros((NROW, H), jnp.float32)
        for t in range(W):
            gx = gx_ref[t * NROW:(t + 1) * NROW, :]
            gh = jnp.dot(h, whu, preferred_element_type=jnp.float32)
            r = sg(gx[:, :H] + gh[:, :H])
            z = sg(gx[:, H:2 * H] + gh[:, H:2 * H])
            nn = jnp.tanh(gx[:, 2 * H:] + r * gh[:, 2 * H:])
            h = jnp.where(lenv > t, nn + z * (h - nn), h)

        g2_ref[...] = (jnp.dot(h, wxc_ref[...],
                               preferred_element_type=jnp.float32) + bc_ref[...])
        whc = whc_ref[...]
        clv = cl_ref[...]

        h2 = jnp.zeros((B, H), jnp.float32)
        for s in range(U + 1):
            gx2 = g2_ref[s * B:(s + 1) * B, :]
            gh2 = jnp.dot(h2, whc, preferred_element_type=jnp.float32)
            r2 = sg(gx2[:, :H] + gh2[:, :H])
            z2 = sg(gx2[:, H:2 * H] + gh2[:, H:2 * H])
            n2 = jnp.tanh(gx2[:, 2 * H:] + r2 * gh2[:, 2 * H:])
            h2 = jnp.where(clv >= s, n2 + z2 * (h2 - n2), h2)
        out_ref[...] = h2

    return pl.pallas_call(
        tc_kernel,
        out_shape=jax.ShapeDtypeStruct((B, H), jnp.float32),
        scratch_shapes=[pltpu.VMEM((W * NROW, 3 * H), jnp.float32),
                        pltpu.VMEM((NROW, 3 * H), jnp.float32)],
    )(x_tm, lens, ctx_len, wx_u, wh_u, b_u, wx_c, wh_c, b_c)


def kernel(contexts, context_utterance_lengths, context_lengths, queries,
           query_lengths, emb, Wx_u, Wh_u, b_u, Wx_c, Wh_c, b_c):
    x_tm, lens = _sc_gather(contexts.reshape(-1), queries.reshape(-1),
                            context_utterance_lengths.reshape(-1),
                            query_lengths, context_lengths, emb)
    return _tc_gru(x_tm, lens.reshape(NROW, 1), context_lengths.reshape(B, 1),
                   Wx_u, Wh_u, b_u.reshape(1, 3 * H),
                   Wx_c, Wh_c, b_c.reshape(1, 3 * H))


# speedup vs baseline: 1.0906x; 1.0906x over previous
import functools

import jax
import jax.numpy as jnp
from jax import lax
from jax.experimental import pallas as pl
from jax.experimental.pallas import tpu as pltpu
from jax.experimental.pallas import tpu_sc as plsc

V = 30000
D = 256
H = 256
B = 16
U = 15
W = 30
NROW = B * (U + 1)
P = NROW * W
NW = 32
RPW = P // NW
NPW = NROW // NW
CHUNK = 80
WPC = CHUNK // NPW
TCH = 6
NCH = W // TCH


def _sc_gather(toks, ctx_len, emb):
    mesh = plsc.VectorSubcoreMesh(core_axis_name="c", subcore_axis_name="s")

    @functools.partial(
        pl.kernel,
        out_type=jax.ShapeDtypeStruct((W, NROW, D), jnp.float32),
        mesh=mesh,
        compiler_params=pltpu.CompilerParams(needs_layout_passes=False),
        scratch_types=[
            pltpu.VMEM((P,), jnp.int32),
            pltpu.VMEM((128,), jnp.int32),
            pltpu.VMEM((RPW,), jnp.int32),
            pltpu.VMEM((RPW, D), jnp.float32),
            pltpu.SemaphoreType.DMA,
            pltpu.SemaphoreType.DMA,
        ],
    )
    def sc_kernel(toks_hbm, len_hbm, emb_hbm, x_hbm,
                  toks_v, len_v, idx_v, rows_v, gsem, osem):
        wid = lax.axis_index("s") * 2 + lax.axis_index("c")
        n0 = wid * NPW
        pltpu.sync_copy(toks_hbm, toks_v)
        pltpu.sync_copy(len_hbm, len_v.at[pl.ds(0, B)])
        lane = lax.iota(jnp.int32, 16)
        us = n0 >> 4
        b = (n0 & 15) + (lane & 7)
        lb = plsc.load_gather(len_v, [b])
        su = jnp.maximum(jnp.where(us < lb, us, us - 1), 0)
        off0 = jnp.where(lb == us,
                         B * U * W + b * W,
                         b * (U * W) + su * W)
        wbase = lane >> 3
        out_descs = []
        for c in range(RPW // CHUNK):
            for i in range(CHUNK // 16):
                w = c * WPC + 2 * i + wbase
                idx_v[pl.ds(c * CHUNK + i * 16, 16)] = plsc.load_gather(
                    toks_v, [off0 + w])
            gd = pltpu.async_copy(
                emb_hbm.at[idx_v.at[pl.ds(c * CHUNK, CHUNK)]],
                rows_v.at[pl.ds(c * CHUNK, CHUNK)],
                gsem,
            )
            if c > 0:
                for w in range((c - 1) * WPC, c * WPC):
                    out_descs.append(pltpu.async_copy(
                        rows_v.at[pl.ds(w * NPW, NPW)],
                        x_hbm.at[w, pl.ds(n0, NPW)],
                        osem,
                    ))
            gd.wait()
        for w in range((RPW // CHUNK - 1) * WPC, W):
            out_descs.append(pltpu.async_copy(
                rows_v.at[pl.ds(w * NPW, NPW)],
                x_hbm.at[w, pl.ds(n0, NPW)],
                osem,
            ))
        for d in out_descs:
            d.wait()

    return sc_kernel(toks, ctx_len, emb)


def _tc_gru(x_tm, cul, ql, ctx_len, wx_u, wh_u, b_u, wx_c, wh_c, b_c):
    def sg(a):
        return 0.5 + 0.5 * jnp.tanh(0.5 * a)

    def col16(row):
        eye = (lax.broadcasted_iota(jnp.int32, (B, B), 0)
               == lax.broadcasted_iota(jnp.int32, (B, B), 1)).astype(jnp.float32)
        return jnp.dot(eye * row, jnp.ones((B, 1), jnp.float32),
                       preferred_element_type=jnp.float32)

    def tc_kernel(x_ref, cul_ref, ql_ref, cl_ref, wxu_ref, whu_ref, bu_ref,
                  wxc_ref, whc_ref, bc_ref, out_ref, h_ref, len_ref):
        i = pl.program_id(0)

        @pl.when(i == 0)
        def _():
            h_ref[...] = jnp.zeros((NROW, H), jnp.float32)
            qlc = col16(ql_ref[...].astype(jnp.float32).reshape(1, B))
            cc = jnp.concatenate(
                [cul_ref[...].astype(jnp.float32), qlc], axis=1)
            rows = lax.broadcasted_iota(jnp.int32, (NROW, B), 0)
            cols = lax.broadcasted_iota(jnp.int32, (NROW, B), 1)
            sb = ((rows & 15) == cols).astype(jnp.float32)
            mu = ((rows >> 4) == cols).astype(jnp.float32)
            len_ref[...] = jnp.dot(
                jnp.dot(sb, cc, preferred_element_type=jnp.float32) * mu,
                jnp.ones((B, 1), jnp.float32),
                preferred_element_type=jnp.float32)

        whu = whu_ref[...]
        lenv = len_ref[...]
        gxc = (jnp.dot(x_ref[...].reshape(TCH * NROW, D), wxu_ref[...],
                       preferred_element_type=jnp.float32) + bu_ref[...])
        h = h_ref[...]
        for tl in range(TCH):
            gx = gxc[tl * NROW:(tl + 1) * NROW, :]
            gh = jnp.dot(h, whu, preferred_element_type=jnp.float32)
            r = sg(gx[:, :H] + gh[:, :H])
            z = sg(gx[:, H:2 * H] + gh[:, H:2 * H])
            nn = jnp.tanh(gx[:, 2 * H:] + r * gh[:, 2 * H:])
            t = i * TCH + tl
            h = jnp.where(lenv > t, nn + z * (h - nn), h)
        h_ref[...] = h

        @pl.when(i == NCH - 1)
        def _():
            g2 = (jnp.dot(h, wxc_ref[...],
                          preferred_element_type=jnp.float32) + bc_ref[...])
            whc = whc_ref[...]
            clv = col16(cl_ref[...].astype(jnp.float32).reshape(1, B))
            h2 = jnp.zeros((B, H), jnp.float32)
            for s in range(U + 1):
                gx2 = g2[s * B:(s + 1) * B, :]
                gh2 = jnp.dot(h2, whc, preferred_element_type=jnp.float32)
                r2 = sg(gx2[:, :H] + gh2[:, :H])
                z2 = sg(gx2[:, H:2 * H] + gh2[:, H:2 * H])
                n2 = jnp.tanh(gx2[:, 2 * H:] + r2 * gh2[:, 2 * H:])
                h2 = jnp.where(clv >= s, n2 + z2 * (h2 - n2), h2)
            out_ref[...] = h2

    full = lambda shape: pl.BlockSpec(shape, lambda i: tuple(0 for _ in shape))
    return pl.pallas_call(
        tc_kernel,
        grid=(NCH,),
        in_specs=[
            pl.BlockSpec((TCH, NROW, D), lambda i: (i, 0, 0)),
            full((B, U)),
            full((B,)),
            full((B,)),
            full((D, 3 * H)),
            full((H, 3 * H)),
            full((3 * H,)),
            full((H, 3 * H)),
            full((H, 3 * H)),
            full((3 * H,)),
        ],
        out_specs=full((B, H)),
        out_shape=jax.ShapeDtypeStruct((B, H), jnp.float32),
        scratch_shapes=[pltpu.VMEM((NROW, H), jnp.float32),
                        pltpu.VMEM((NROW, 1), jnp.float32)],
    )(x_tm, cul, ql, ctx_len, wx_u, wh_u, b_u, wx_c, wh_c, b_c)


def kernel(contexts, context_utterance_lengths, context_lengths, queries,
           query_lengths, emb, Wx_u, Wh_u, b_u, Wx_c, Wh_c, b_c):
    toks = jnp.concatenate([contexts.reshape(-1), queries.reshape(-1)])
    x_tm = _sc_gather(toks, context_lengths, emb)
    return _tc_gru(x_tm, context_utterance_lengths, query_lengths,
                   context_lengths, Wx_u, Wh_u, b_u, Wx_c, Wh_c, b_c)
